# baseline (speedup 1.0000x reference)
import functools

import jax
import jax.numpy as jnp
from jax import lax
from jax.experimental import pallas as pl
from jax.experimental.pallas import tpu as pltpu
from jax.experimental.pallas import tpu_sc as plsc

N_PROPS = 528
D_HALF = 256
D_FULL = 512
PE_ROWS = 64
ROWS_PER_CHUNK = 16
N_CHUNKS = N_PROPS // ROWS_PER_CHUNK

_SC_MESH = plsc.VectorSubcoreMesh(core_axis_name="c", subcore_axis_name="s")


@functools.partial(
    pl.kernel,
    mesh=_SC_MESH,
    out_type=jax.ShapeDtypeStruct((N_PROPS, D_HALF), jnp.float32),
    scratch_types=[
        pltpu.VMEM((ROWS_PER_CHUNK,), jnp.int32),
        pltpu.VMEM((ROWS_PER_CHUNK, D_HALF), jnp.float32),
        pltpu.SemaphoreType.DMA,
    ],
)
def _sc_gather_bias(table_hbm, idx_hbm, out_hbm, idx_v, rows_v, sem):
    wid = lax.axis_index("s") * 2 + lax.axis_index("c")

    def do_chunk(cid):
        base = cid * ROWS_PER_CHUNK
        pltpu.sync_copy(idx_hbm.at[pl.ds(base, ROWS_PER_CHUNK)], idx_v)
        pltpu.async_copy(table_hbm.at[idx_v], rows_v, sem).wait()
        pltpu.sync_copy(rows_v, out_hbm.at[pl.ds(base, ROWS_PER_CHUNK)])

    do_chunk(wid)

    @pl.when(wid < N_CHUNKS - 32)
    def _():
        do_chunk(32 + wid)


def _add_body(x_ref, b_ref, o_ref):
    b = b_ref[...]
    o_ref[:, :, :D_HALF] = x_ref[:, :, :D_HALF] + b[None]
    o_ref[:, :, D_HALF:] = x_ref[:, :, D_HALF:] + b[None]


def kernel(x, pe, props):
    bsz = x.shape[0]
    table = pe.reshape(PE_ROWS, D_HALF)
    idx = props[:, 0]

    bias = _sc_gather_bias(table, idx)

    b_blk = 8
    out = pl.pallas_call(
        _add_body,
        grid=(bsz // b_blk,),
        in_specs=[
            pl.BlockSpec((b_blk, N_PROPS, D_FULL), lambda i: (i, 0, 0)),
            pl.BlockSpec((N_PROPS, D_HALF), lambda i: (0, 0)),
        ],
        out_specs=pl.BlockSpec((b_blk, N_PROPS, D_FULL), lambda i: (i, 0, 0)),
        out_shape=jax.ShapeDtypeStruct(x.shape, x.dtype),
        compiler_params=pltpu.CompilerParams(
            dimension_semantics=("parallel",),
        ),
    )(x, bias)
    return out


# speedup vs baseline: 1.0044x; 1.0044x over previous
import functools

import jax
import jax.numpy as jnp
from jax import lax
from jax.experimental import pallas as pl
from jax.experimental.pallas import tpu as pltpu
from jax.experimental.pallas import tpu_sc as plsc

N_PROPS = 528
D_HALF = 256
D_FULL = 512
PE_ROWS = 64
ROWS_PER_CHUNK = 24
N_CHUNKS = N_PROPS // ROWS_PER_CHUNK

_SC_MESH = plsc.VectorSubcoreMesh(core_axis_name="c", subcore_axis_name="s")


@functools.partial(
    pl.kernel,
    mesh=_SC_MESH,
    out_type=jax.ShapeDtypeStruct((N_PROPS, D_HALF), jnp.float32),
    scratch_types=[
        pltpu.VMEM((ROWS_PER_CHUNK,), jnp.int32),
        pltpu.VMEM((ROWS_PER_CHUNK, D_HALF), jnp.float32),
        pltpu.SemaphoreType.DMA,
    ],
)
def _sc_gather_bias(table_hbm, idx_hbm, out_hbm, idx_v, rows_v, sem):
    wid = lax.axis_index("s") * 2 + lax.axis_index("c")

    def do_chunk(cid):
        base = cid * ROWS_PER_CHUNK
        pltpu.sync_copy(idx_hbm.at[pl.ds(base, ROWS_PER_CHUNK)], idx_v)
        pltpu.async_copy(table_hbm.at[idx_v], rows_v, sem).wait()
        pltpu.sync_copy(rows_v, out_hbm.at[pl.ds(base, ROWS_PER_CHUNK)])

    @pl.when(wid < N_CHUNKS)
    def _():
        do_chunk(wid)


def _add_body(x_ref, b_ref, o_ref):
    b = b_ref[...]
    o_ref[:, :, :D_HALF] = x_ref[:, :, :D_HALF] + b[None]
    o_ref[:, :, D_HALF:] = x_ref[:, :, D_HALF:] + b[None]


def kernel(x, pe, props):
    bsz = x.shape[0]
    table = pe.reshape(PE_ROWS, D_HALF)
    idx = props[:, 0]

    bias = _sc_gather_bias(table, idx)

    b_blk = 8
    out = pl.pallas_call(
        _add_body,
        grid=(bsz // b_blk,),
        in_specs=[
            pl.BlockSpec((b_blk, N_PROPS, D_FULL), lambda i: (i, 0, 0)),
            pl.BlockSpec((N_PROPS, D_HALF), lambda i: (0, 0)),
        ],
        out_specs=pl.BlockSpec((b_blk, N_PROPS, D_FULL), lambda i: (i, 0, 0)),
        out_shape=jax.ShapeDtypeStruct(x.shape, x.dtype),
        compiler_params=pltpu.CompilerParams(
            dimension_semantics=("parallel",),
        ),
    )(x, bias)
    return out
